# baseline (speedup 1.0000x reference)
import jax
import jax.numpy as jnp
from jax.experimental import pallas as pl
from jax.experimental.pallas import tpu as pltpu
from jax.experimental.pallas import tpu_sc as plsc

B = 16384
D = 64

OFF_H, OFF_W, OFF_D, OFF_P = 0, 24, 31, 41
NROWS = 48

NR = 8400
NRP = 8448

B_SC = 4096
B_TC = B - B_SC
BB = 4096
NB = B_TC // BB

NC, NS = 2, 16
NW = NC * NS
BPW = B_SC // NW


def _fused_centered(ht_ref, wt_ref, dt_ref, pt_ref, W_ref, b_ref):
    fused = jnp.zeros((NROWS, 64), jnp.float32)
    for f, (off, ref) in enumerate(((OFF_H, ht_ref), (OFF_W, wt_ref),
                                    (OFF_D, dt_ref), (OFF_P, pt_ref))):
        n = ref.shape[0]
        rsel = jax.lax.broadcasted_iota(jnp.int32, (NROWS, n), 0) - off
        csel = jax.lax.broadcasted_iota(jnp.int32, (NROWS, n), 1)
        S = jnp.where(rsel == csel, 1.0, 0.0)
        placed = jnp.dot(S, ref[...], preferred_element_type=jnp.float32)
        fused = fused + jnp.dot(placed, W_ref[pl.ds(64 * f, 64), :],
                                preferred_element_type=jnp.float32)
    fused = fused - jnp.mean(fused, axis=1, keepdims=True)
    ri = jax.lax.broadcasted_iota(jnp.int32, (NROWS, 64), 0)
    bc = b_ref[...] - jnp.mean(b_ref[...])
    is_p = (ri >= OFF_P) & (ri < 46)
    fused = jnp.where(is_p, fused + bc, fused)
    hi16 = fused.astype(jnp.bfloat16)
    lo16 = (fused - hi16.astype(jnp.float32)).astype(jnp.bfloat16)
    return hi16, lo16


def _norm_tail(xc, g_ref, be_ref):
    sq = xc * xc
    ones = jnp.full((64, 64), 1.0 / 64.0, jnp.float32)
    var = jnp.dot(sq, ones, preferred_element_type=jnp.float32)
    rs = jax.lax.rsqrt(var + 1e-5)
    y = xc * (rs * g_ref[...]) + be_ref[...]
    return jnp.maximum(y, 0.0)


def _encode_block(h_ref, w_ref, d_ref, p_ref, ht_ref, wt_ref, dt_ref, pt_ref,
                  W_ref, b_ref, g_ref, be_ref, sc_ref, out_ref, hi_s, lo_s):
    @pl.when(pl.program_id(0) >= NB)
    def _():
        out_ref[...] = sc_ref[:, pl.ds(0, D)]

    @pl.when(pl.program_id(0) == 0)
    def _():
        h16, l16 = _fused_centered(ht_ref, wt_ref, dt_ref, pt_ref,
                                   W_ref, b_ref)
        hi_s[...] = h16
        lo_s[...] = l16

    @pl.when(pl.program_id(0) < NB)
    def _():
        hi16 = hi_s[...]
        lo16 = lo_s[...]
        ci = jax.lax.broadcasted_iota(jnp.int32,
                                      (NROWS, BB), 0).astype(jnp.bfloat16)
        one = jnp.ones((), jnp.bfloat16)
        zero = jnp.zeros((), jnp.bfloat16)
        mh = (jnp.where(ci == h_ref[0].astype(jnp.bfloat16), one, zero)
              + jnp.where((ci - OFF_W) == w_ref[0].astype(jnp.bfloat16),
                          one, zero)
              + jnp.where((ci - OFF_D) == d_ref[0].astype(jnp.bfloat16),
                          one, zero)
              + jnp.where((ci - OFF_P) == p_ref[0].astype(jnp.bfloat16),
                          one, zero))
        dn = (((0,), (0,)), ((), ()))
        xc = (jax.lax.dot_general(mh, hi16, dn,
                                  preferred_element_type=jnp.float32)
              + jax.lax.dot_general(mh, lo16, dn,
                                    preferred_element_type=jnp.float32))
        out_ref[...] = _norm_tail(xc, g_ref, be_ref)


def _encode_tc(hour, weekday, device, platform, ht, wt, dt, pt, W, b,
               gamma, beta, out_sc, interpret=False):
    idx3 = lambda a: a.reshape(B // BB, 1, BB)
    idx_spec = pl.BlockSpec((1, 1, BB), lambda i: (i, 0, 0))
    full = lambda shape: pl.BlockSpec(shape, lambda i: tuple(0 for _ in shape))
    sc_spec = pl.BlockSpec((BB, 2 * D), lambda i: (jnp.maximum(i - NB, 0), 0))
    return pl.pallas_call(
        _encode_block,
        grid=(B // BB,),
        in_specs=[idx_spec, idx_spec, idx_spec, idx_spec,
                  full(ht.shape), full(wt.shape), full(dt.shape),
                  full(pt.shape), full((256, 64)), full((1, 64)),
                  full((1, 64)), full((1, 64)), sc_spec],
        out_specs=pl.BlockSpec((BB, 64), lambda i: (i, 0)),
        out_shape=jax.ShapeDtypeStruct((B, D), jnp.float32),
        scratch_shapes=[pltpu.VMEM((NROWS, 64), jnp.bfloat16),
                        pltpu.VMEM((NROWS, 64), jnp.bfloat16)],
        interpret=interpret,
    )(idx3(hour), idx3(weekday), idx3(device), idx3(platform),
      ht, wt, dt, pt, W, b.reshape(1, D), gamma.reshape(1, D),
      beta.reshape(1, D), out_sc)


def _splice_block(tc_ref, sc_ref, out_ref):
    del tc_ref
    out_ref[...] = sc_ref[:, pl.ds(0, D)]


def _splice(out_tc, out_sc):
    return pl.pallas_call(
        _splice_block,
        grid=(B_SC // BB,),
        in_specs=[pl.BlockSpec((8, D), lambda i: (0, 0)),
                  pl.BlockSpec((BB, 2 * D), lambda i: (i, 0))],
        out_specs=pl.BlockSpec((BB, D), lambda i: (NB + i, 0)),
        out_shape=jax.ShapeDtypeStruct((B, D), jnp.float32),
        input_output_aliases={0: 0},
    )(out_tc, out_sc)


NRB = 2816


def _build_block(ht_ref, wt_ref, dt_ref, pt_ref, W_ref, b_ref, g_ref,
                 be_ref, tab_ref, hi_s, lo_s):
    i = pl.program_id(0)

    @pl.when(i == 0)
    def _():
        h16, l16 = _fused_centered(ht_ref, wt_ref, dt_ref, pt_ref,
                                   W_ref, b_ref)
        hi_s[...] = h16
        lo_s[...] = l16

    hi16 = hi_s[...]
    lo16 = lo_s[...]
    r = jax.lax.broadcasted_iota(jnp.int32, (8, NRB), 1) + i * NRB
    q350 = r // 350
    q50 = r // 50
    q5 = r // 5
    to16 = lambda a: a[0:1].astype(jnp.bfloat16)
    hr = to16(q350)
    wr = to16(q50 - 7 * q350 + OFF_W)
    dr = to16(q5 - 10 * q50 + OFF_D)
    pr = to16(r - 5 * q5 + OFF_P)
    ci = jax.lax.broadcasted_iota(jnp.int32, (NROWS, NRB), 0).astype(jnp.bfloat16)
    one = jnp.ones((), jnp.bfloat16)
    zero = jnp.zeros((), jnp.bfloat16)
    mh = (jnp.where(ci == hr, one, zero) + jnp.where(ci == wr, one, zero)
          + jnp.where(ci == dr, one, zero) + jnp.where(ci == pr, one, zero))
    dn = (((0,), (0,)), ((), ()))
    xc = (jax.lax.dot_general(mh, hi16, dn, preferred_element_type=jnp.float32)
          + jax.lax.dot_general(mh, lo16, dn,
                                preferred_element_type=jnp.float32))
    y = _norm_tail(xc, g_ref, be_ref)
    tab_ref[:, pl.ds(0, D)] = y
    tab_ref[:, pl.ds(D, D)] = y


def _build_table(ht, wt, dt, pt, W, b, gamma, beta, interpret=False):
    full = lambda shape: pl.BlockSpec(shape, lambda i: tuple(0 for _ in shape))
    return pl.pallas_call(
        _build_block,
        grid=(NRP // NRB,),
        in_specs=[full(ht.shape), full(wt.shape), full(dt.shape),
                  full(pt.shape), full((256, 64)), full((1, 64)),
                  full((1, 64)), full((1, 64))],
        out_specs=pl.BlockSpec((NRB, 2 * D), lambda i: (i, 0)),
        out_shape=jax.ShapeDtypeStruct((NRP, 2 * D), jnp.float32),
        scratch_shapes=[pltpu.VMEM((NROWS, 64), jnp.bfloat16),
                        pltpu.VMEM((NROWS, 64), jnp.bfloat16)],
        interpret=interpret,
    )(ht, wt, dt, pt, W, b.reshape(1, D), gamma.reshape(1, D),
      beta.reshape(1, D))


def _vector_mesh():
    return plsc.VectorSubcoreMesh(core_axis_name="core",
                                  subcore_axis_name="subcore")


def _sc_gather(table, hour, weekday, device, platform):
    @pl.kernel(out_type=jax.ShapeDtypeStruct((B_SC, 2 * D), jnp.float32),
               mesh=_vector_mesh(),
               scratch_types=[pltpu.VMEM((BPW,), jnp.int32),
                              pltpu.VMEM((BPW,), jnp.int32),
                              pltpu.VMEM((BPW, 2 * D), jnp.float32),
                              pltpu.SemaphoreType.DMA])
    def k(tab_hbm, h_hbm, w_hbm, d_hbm, p_hbm, o_hbm,
          idx_v, tmp_v, rows_v, sem):
        wid = (jax.lax.axis_index("subcore") * NC
               + jax.lax.axis_index("core"))
        base = B_TC + wid * BPW
        pltpu.sync_copy(h_hbm.at[pl.ds(base, BPW)], idx_v)
        pltpu.sync_copy(w_hbm.at[pl.ds(base, BPW)], tmp_v)

        @pl.loop(0, BPW, step=16)
        def _(j):
            s = pl.ds(j, 16)
            idx_v.at[s][...] = idx_v.at[s][...] * 7 + tmp_v.at[s][...]

        pltpu.sync_copy(d_hbm.at[pl.ds(base, BPW)], tmp_v)

        @pl.loop(0, BPW, step=16)
        def _(j):
            s = pl.ds(j, 16)
            idx_v.at[s][...] = idx_v.at[s][...] * 10 + tmp_v.at[s][...]

        pltpu.sync_copy(p_hbm.at[pl.ds(base, BPW)], tmp_v)

        @pl.loop(0, BPW, step=16)
        def _(j):
            s = pl.ds(j, 16)
            idx_v.at[s][...] = idx_v.at[s][...] * 5 + tmp_v.at[s][...]

        pltpu.async_copy(tab_hbm.at[idx_v], rows_v, sem).wait()
        pltpu.sync_copy(rows_v, o_hbm.at[pl.ds(wid * BPW, BPW)])

    return k(table, hour, weekday, device, platform)


def kernel(hour, weekday, device, platform, hour_table, weekday_table,
           device_table, platform_table, W, b, gamma, beta):
    table = _build_table(hour_table, weekday_table, device_table,
                         platform_table, W, b, gamma, beta)
    out_sc = _sc_gather(table, hour, weekday, device, platform)
    return _encode_tc(hour, weekday, device, platform, hour_table,
                      weekday_table, device_table, platform_table,
                      W, b, gamma, beta, out_sc)


# speedup vs baseline: 1.0875x; 1.0875x over previous
import jax
import jax.numpy as jnp
from jax.experimental import pallas as pl
from jax.experimental.pallas import tpu as pltpu
from jax.experimental.pallas import tpu_sc as plsc

B = 16384
D = 64

OFF_H, OFF_W, OFF_D, OFF_P = 0, 24, 31, 41
NROWS = 48

NR = 8400
NRP = 8448

B_SC = 4096
B_TC = B - B_SC
BB = 4096
NB = B_TC // BB

NC, NS = 2, 16
NW = NC * NS
BPW = B_SC // NW


def _fused_centered(ht_ref, wt_ref, dt_ref, pt_ref, W_ref, b_ref):
    fused = jnp.zeros((NROWS, 64), jnp.float32)
    for f, (off, ref) in enumerate(((OFF_H, ht_ref), (OFF_W, wt_ref),
                                    (OFF_D, dt_ref), (OFF_P, pt_ref))):
        n = ref.shape[0]
        rsel = jax.lax.broadcasted_iota(jnp.int32, (NROWS, n), 0) - off
        csel = jax.lax.broadcasted_iota(jnp.int32, (NROWS, n), 1)
        S = jnp.where(rsel == csel, 1.0, 0.0)
        placed = jnp.dot(S, ref[...], preferred_element_type=jnp.float32)
        fused = fused + jnp.dot(placed, W_ref[pl.ds(64 * f, 64), :],
                                preferred_element_type=jnp.float32)
    fused = fused - jnp.mean(fused, axis=1, keepdims=True)
    ri = jax.lax.broadcasted_iota(jnp.int32, (NROWS, 64), 0)
    bc = b_ref[...] - jnp.mean(b_ref[...])
    is_p = (ri >= OFF_P) & (ri < 46)
    fused = jnp.where(is_p, fused + bc, fused)
    hi16 = fused.astype(jnp.bfloat16)
    lo16 = (fused - hi16.astype(jnp.float32)).astype(jnp.bfloat16)
    return hi16, lo16


def _norm_tail(xc, g_ref, be_ref):
    sq = xc * xc
    ones = jnp.full((64, 64), 1.0 / 64.0, jnp.float32)
    var = jnp.dot(sq, ones, preferred_element_type=jnp.float32)
    rs = jax.lax.rsqrt(var + 1e-5)
    y = xc * (rs * g_ref[...]) + be_ref[...]
    return jnp.maximum(y, 0.0)


def _encode_block(h_ref, w_ref, d_ref, p_ref, ht_ref, wt_ref, dt_ref, pt_ref,
                  W_ref, b_ref, g_ref, be_ref, out_ref, hi_s, lo_s):
    @pl.when(pl.program_id(0) == 0)
    def _():
        h16, l16 = _fused_centered(ht_ref, wt_ref, dt_ref, pt_ref,
                                   W_ref, b_ref)
        hi_s[...] = h16
        lo_s[...] = l16

    hi16 = hi_s[...]
    lo16 = lo_s[...]
    ci = jax.lax.broadcasted_iota(jnp.int32, (NROWS, BB), 0).astype(jnp.bfloat16)
    one = jnp.ones((), jnp.bfloat16)
    zero = jnp.zeros((), jnp.bfloat16)
    mh = (jnp.where(ci == h_ref[0].astype(jnp.bfloat16), one, zero)
          + jnp.where((ci - OFF_W) == w_ref[0].astype(jnp.bfloat16), one, zero)
          + jnp.where((ci - OFF_D) == d_ref[0].astype(jnp.bfloat16), one, zero)
          + jnp.where((ci - OFF_P) == p_ref[0].astype(jnp.bfloat16), one, zero))
    dn = (((0,), (0,)), ((), ()))
    xc = (jax.lax.dot_general(mh, hi16, dn, preferred_element_type=jnp.float32)
          + jax.lax.dot_general(mh, lo16, dn,
                                preferred_element_type=jnp.float32))
    out_ref[...] = _norm_tail(xc, g_ref, be_ref)


def _encode_tc(hour, weekday, device, platform, ht, wt, dt, pt, W, b,
               gamma, beta, interpret=False):
    idx3 = lambda a: a.reshape(B // BB, 1, BB)
    idx_spec = pl.BlockSpec((1, 1, BB), lambda i: (i, 0, 0))
    full = lambda shape: pl.BlockSpec(shape, lambda i: tuple(0 for _ in shape))
    return pl.pallas_call(
        _encode_block,
        grid=(NB,),
        in_specs=[idx_spec, idx_spec, idx_spec, idx_spec,
                  full(ht.shape), full(wt.shape), full(dt.shape),
                  full(pt.shape), full((256, 64)), full((1, 64)),
                  full((1, 64)), full((1, 64))],
        out_specs=pl.BlockSpec((BB, 64), lambda i: (i, 0)),
        out_shape=jax.ShapeDtypeStruct((B, D), jnp.float32),
        scratch_shapes=[pltpu.VMEM((NROWS, 64), jnp.bfloat16),
                        pltpu.VMEM((NROWS, 64), jnp.bfloat16)],
        interpret=interpret,
    )(idx3(hour), idx3(weekday), idx3(device), idx3(platform),
      ht, wt, dt, pt, W, b.reshape(1, D), gamma.reshape(1, D),
      beta.reshape(1, D))


def _splice_block(tc_ref, sc_ref, out_ref):
    del tc_ref
    out_ref[...] = sc_ref[:, pl.ds(0, D)]


def _splice(out_tc, out_sc):
    return pl.pallas_call(
        _splice_block,
        grid=(B_SC // BB,),
        in_specs=[pl.BlockSpec((8, D), lambda i: (0, 0)),
                  pl.BlockSpec((BB, 2 * D), lambda i: (i, 0))],
        out_specs=pl.BlockSpec((BB, D), lambda i: (NB + i, 0)),
        out_shape=jax.ShapeDtypeStruct((B, D), jnp.float32),
        input_output_aliases={0: 0},
    )(out_tc, out_sc)


NRB = 2816


def _build_block(ht_ref, wt_ref, dt_ref, pt_ref, W_ref, b_ref, g_ref,
                 be_ref, tab_ref, hi_s, lo_s):
    i = pl.program_id(0)

    @pl.when(i == 0)
    def _():
        h16, l16 = _fused_centered(ht_ref, wt_ref, dt_ref, pt_ref,
                                   W_ref, b_ref)
        hi_s[...] = h16
        lo_s[...] = l16

    hi16 = hi_s[...]
    lo16 = lo_s[...]
    r = jax.lax.broadcasted_iota(jnp.int32, (8, NRB), 1) + i * NRB
    q350 = r // 350
    q50 = r // 50
    q5 = r // 5
    to16 = lambda a: a[0:1].astype(jnp.bfloat16)
    hr = to16(q350)
    wr = to16(q50 - 7 * q350 + OFF_W)
    dr = to16(q5 - 10 * q50 + OFF_D)
    pr = to16(r - 5 * q5 + OFF_P)
    ci = jax.lax.broadcasted_iota(jnp.int32, (NROWS, NRB), 0).astype(jnp.bfloat16)
    one = jnp.ones((), jnp.bfloat16)
    zero = jnp.zeros((), jnp.bfloat16)
    mh = (jnp.where(ci == hr, one, zero) + jnp.where(ci == wr, one, zero)
          + jnp.where(ci == dr, one, zero) + jnp.where(ci == pr, one, zero))
    dn = (((0,), (0,)), ((), ()))
    xc = (jax.lax.dot_general(mh, hi16, dn, preferred_element_type=jnp.float32)
          + jax.lax.dot_general(mh, lo16, dn,
                                preferred_element_type=jnp.float32))
    y = _norm_tail(xc, g_ref, be_ref)
    tab_ref[:, pl.ds(0, D)] = y
    tab_ref[:, pl.ds(D, D)] = y


def _build_table(ht, wt, dt, pt, W, b, gamma, beta, interpret=False):
    full = lambda shape: pl.BlockSpec(shape, lambda i: tuple(0 for _ in shape))
    return pl.pallas_call(
        _build_block,
        grid=(NRP // NRB,),
        in_specs=[full(ht.shape), full(wt.shape), full(dt.shape),
                  full(pt.shape), full((256, 64)), full((1, 64)),
                  full((1, 64)), full((1, 64))],
        out_specs=pl.BlockSpec((NRB, 2 * D), lambda i: (i, 0)),
        out_shape=jax.ShapeDtypeStruct((NRP, 2 * D), jnp.float32),
        scratch_shapes=[pltpu.VMEM((NROWS, 64), jnp.bfloat16),
                        pltpu.VMEM((NROWS, 64), jnp.bfloat16)],
        interpret=interpret,
    )(ht, wt, dt, pt, W, b.reshape(1, D), gamma.reshape(1, D),
      beta.reshape(1, D))


def _vector_mesh():
    return plsc.VectorSubcoreMesh(core_axis_name="core",
                                  subcore_axis_name="subcore")


def _sc_gather(table, hour, weekday, device, platform):
    @pl.kernel(out_type=jax.ShapeDtypeStruct((B_SC, 2 * D), jnp.float32),
               mesh=_vector_mesh(),
               scratch_types=[pltpu.VMEM((BPW,), jnp.int32),
                              pltpu.VMEM((BPW,), jnp.int32),
                              pltpu.VMEM((BPW, 2 * D), jnp.float32),
                              pltpu.SemaphoreType.DMA])
    def k(tab_hbm, h_hbm, w_hbm, d_hbm, p_hbm, o_hbm,
          idx_v, tmp_v, rows_v, sem):
        wid = (jax.lax.axis_index("subcore") * NC
               + jax.lax.axis_index("core"))
        base = B_TC + wid * BPW
        pltpu.sync_copy(h_hbm.at[pl.ds(base, BPW)], idx_v)
        pltpu.sync_copy(w_hbm.at[pl.ds(base, BPW)], tmp_v)

        @pl.loop(0, BPW, step=16)
        def _(j):
            s = pl.ds(j, 16)
            idx_v.at[s][...] = idx_v.at[s][...] * 7 + tmp_v.at[s][...]

        pltpu.sync_copy(d_hbm.at[pl.ds(base, BPW)], tmp_v)

        @pl.loop(0, BPW, step=16)
        def _(j):
            s = pl.ds(j, 16)
            idx_v.at[s][...] = idx_v.at[s][...] * 10 + tmp_v.at[s][...]

        pltpu.sync_copy(p_hbm.at[pl.ds(base, BPW)], tmp_v)

        @pl.loop(0, BPW, step=16)
        def _(j):
            s = pl.ds(j, 16)
            idx_v.at[s][...] = idx_v.at[s][...] * 5 + tmp_v.at[s][...]

        pltpu.async_copy(tab_hbm.at[idx_v], rows_v, sem).wait()
        pltpu.sync_copy(rows_v, o_hbm.at[pl.ds(wid * BPW, BPW)])

    return k(table, hour, weekday, device, platform)


def kernel(hour, weekday, device, platform, hour_table, weekday_table,
           device_table, platform_table, W, b, gamma, beta):
    table = _build_table(hour_table, weekday_table, device_table,
                         platform_table, W, b, gamma, beta)
    out_sc = _sc_gather(table, hour, weekday, device, platform)
    out_tc = _encode_tc(hour, weekday, device, platform, hour_table,
                        weekday_table, device_table, platform_table,
                        W, b, gamma, beta)
    return _splice(out_tc, out_sc)
